# baseline (speedup 1.0000x reference)
import functools

import jax
import jax.numpy as jnp
from jax import lax
from jax.experimental import pallas as pl
from jax.experimental.pallas import tpu as pltpu
from jax.experimental.pallas import tpu_sc as plsc

U = 10000
V = 10000
IN = 256
S = 5
H0 = 500
H1 = 75
C = 100
SIDE = 128
FH = 64
NB = 2
NC = 5
E = 160000
P = 100000

CP = 128
H0P = S * CP
H1P = 128

BR = 1000
f32 = jnp.float32

SC_NC = 2
SC_NS = 16


def _pre_body(x_ref, side_ref, wpad_ref, wf_ref, bf_ref, tmp_ref, feat_ref):
    tmp_ref[...] = jnp.dot(x_ref[...], wpad_ref[...],
                           preferred_element_type=f32)
    f = jnp.dot(side_ref[...], wf_ref[...], preferred_element_type=f32)
    feat_ref[...] = jnp.maximum(f + bf_ref[...], 0.0)


def _pre(x, side, wpad, wf, bf2d):
    n = x.shape[0]
    return pl.pallas_call(
        _pre_body,
        grid=(n // BR,),
        in_specs=[
            pl.BlockSpec((BR, IN), lambda i: (i, 0)),
            pl.BlockSpec((BR, SIDE), lambda i: (i, 0)),
            pl.BlockSpec((IN, H0P), lambda i: (0, 0)),
            pl.BlockSpec((SIDE, FH), lambda i: (0, 0)),
            pl.BlockSpec((1, FH), lambda i: (0, 0)),
        ],
        out_specs=[
            pl.BlockSpec((BR, H0P), lambda i: (i, 0)),
            pl.BlockSpec((BR, FH), lambda i: (i, 0)),
        ],
        out_shape=[
            jax.ShapeDtypeStruct((n, H0P), f32),
            jax.ShapeDtypeStruct((n, FH), f32),
        ],
    )(x, side, wpad, wf, bf2d)


def _post_u_body(z_ref, feat_ref, wd_ref, wdf_ref, p0_ref, p1_ref, out_ref):
    g = jnp.maximum(z_ref[...], 0.0)
    emb = (jnp.dot(g, wd_ref[...], preferred_element_type=f32)
           + jnp.dot(feat_ref[...], wdf_ref[...], preferred_element_type=f32))
    out_ref[:, :H1P] = jnp.dot(emb, p0_ref[...], preferred_element_type=f32)
    out_ref[:, H1P:] = jnp.dot(emb, p1_ref[...], preferred_element_type=f32)


def _post_u(z, feat, wd, wdf, p0, p1):
    return pl.pallas_call(
        _post_u_body,
        grid=(U // BR,),
        in_specs=[
            pl.BlockSpec((BR, H0P), lambda i: (i, 0)),
            pl.BlockSpec((BR, FH), lambda i: (i, 0)),
            pl.BlockSpec((H0P, H1P), lambda i: (0, 0)),
            pl.BlockSpec((FH, H1P), lambda i: (0, 0)),
            pl.BlockSpec((H1P, H1P), lambda i: (0, 0)),
            pl.BlockSpec((H1P, H1P), lambda i: (0, 0)),
        ],
        out_specs=[pl.BlockSpec((BR, 2 * H1P), lambda i: (i, 0))],
        out_shape=[jax.ShapeDtypeStruct((U, 2 * H1P), f32)],
    )(z, feat, wd, wdf, p0, p1)[0]


def _post_v_body(z_ref, feat_ref, wd_ref, wdf_ref, out_ref):
    g = jnp.maximum(z_ref[...], 0.0)
    out_ref[...] = (jnp.dot(g, wd_ref[...], preferred_element_type=f32)
                    + jnp.dot(feat_ref[...], wdf_ref[...],
                              preferred_element_type=f32))


def _post_v(z, feat, wd, wdf):
    return pl.pallas_call(
        _post_v_body,
        grid=(V // BR,),
        in_specs=[
            pl.BlockSpec((BR, H0P), lambda i: (i, 0)),
            pl.BlockSpec((BR, FH), lambda i: (i, 0)),
            pl.BlockSpec((H0P, H1P), lambda i: (0, 0)),
            pl.BlockSpec((FH, H1P), lambda i: (0, 0)),
        ],
        out_specs=[pl.BlockSpec((BR, H1P), lambda i: (i, 0))],
        out_shape=[jax.ShapeDtypeStruct((V, H1P), f32)],
    )(z, feat, wd, wdf)[0]


P_PAD = 102400
DBR = 1024


def _dec_body(uh_ref, vb_ref, lab_ref, ac_ref, out_ref, loss_ref):
    i = pl.program_id(0)
    uh = uh_ref[...]
    vb = vb_ref[...]
    d0 = jnp.sum(uh[:, :H1P] * vb, axis=1, keepdims=True)
    d1 = jnp.sum(uh[:, H1P:] * vb, axis=1, keepdims=True)
    logits = d0 * ac_ref[0:1, :] + d1 * ac_ref[1:2, :]
    lane = lax.broadcasted_iota(jnp.int32, logits.shape, 1)
    valid = lane < NC
    masked = jnp.where(valid, logits, -1e30)
    m = jnp.max(masked, axis=1, keepdims=True)
    se = jnp.sum(jnp.where(valid, jnp.exp(masked - m), 0.0),
                 axis=1, keepdims=True)
    lse = m + jnp.log(se)
    picked = jnp.sum(jnp.where(lane == lab_ref[...], logits, 0.0),
                     axis=1, keepdims=True)
    out_ref[...] = logits[:, :8]

    @pl.when(i == 0)
    def _():
        loss_ref[...] = jnp.zeros_like(loss_ref)

    row = i * DBR + lax.broadcasted_iota(jnp.int32, (DBR, 1), 0)
    contrib = jnp.where(row < P, lse - picked, 0.0)
    loss_ref[...] += jnp.sum(contrib)[None, None] / P


def _decode(uh, vb, lab2d, ac_pad):
    return pl.pallas_call(
        _dec_body,
        grid=(P_PAD // DBR,),
        in_specs=[
            pl.BlockSpec((DBR, 2 * H1P), lambda i: (i, 0)),
            pl.BlockSpec((DBR, H1P), lambda i: (i, 0)),
            pl.BlockSpec((DBR, 1), lambda i: (i, 0)),
            pl.BlockSpec((8, H1P), lambda i: (0, 0)),
        ],
        out_specs=[
            pl.BlockSpec((DBR, 8), lambda i: (i, 0)),
            pl.BlockSpec((1, 1), lambda i: (0, 0)),
        ],
        out_shape=[
            jax.ShapeDtypeStruct((P_PAD, 8), f32),
            jax.ShapeDtypeStruct((1, 1), f32),
        ],
    )(uh, vb, lab2d, ac_pad)


EPAD = 163840
EPT = EPAD // SC_NS
SEGA = 12504
SEGB = 12496
SPROWS = 12512
DUMMY = 12504
CCAP = EPT + 128
ZROWS = 8
ECHK = 80
WBT = 784


def _edge_agg(tmp_u, tmp_v, epk, val16, zrs):
    @functools.partial(
        pl.kernel,
        mesh=plsc.VectorSubcoreMesh(core_axis_name="c", subcore_axis_name="s"),
        out_type=[
            jax.ShapeDtypeStruct((U * S, CP), f32),
            jax.ShapeDtypeStruct((V * S, CP), f32),
        ],
        scratch_types=[
            pltpu.VMEM((3 * ECHK,), jnp.int32),
            pltpu.VMEM((3 * ECHK,), jnp.int32),
            pltpu.VMEM((ECHK, CP), f32),
            pltpu.VMEM((ECHK, 16), f32),
            pltpu.VMEM((ECHK,), jnp.int32),
            pltpu.VMEM((ECHK,), jnp.int32),
            pltpu.VMEM_SHARED((SPROWS, CP), f32),
            pltpu.SemaphoreType.DMA,
            pltpu.SemaphoreType.DMA,
            pltpu.SemaphoreType.DMA,
        ],
    )
    def ek(epk_hbm, v16_hbm, z_hbm, tu_hbm, tv_hbm, zu_hbm, zv_hbm,
           epk0, epk1, rows_vm, v16_vm, gstage, sstage, seg, sem,
           seme0, seme1):
        epk_vm = (epk0, epk1)
        seme = (seme0, seme1)
        c = lax.axis_index("c")
        s = lax.axis_index("s")
        ebase = s * EPT
        nch = EPT // ECHK

        for uside, k in ((True, 0), (True, 1), (False, 0), (False, 1)):
            table = tv_hbm if uside else tu_hbm
            zout = zu_hbm if uside else zv_hbm
            segsz = SEGA if k == 0 else SEGB
            lo = pl.multiple_of(c * 25000 + k * SEGA, 8)

            @pl.when(s < SC_NS - 1)
            def _():
                pltpu.sync_copy(z_hbm, seg.at[pl.ds(s * WBT, WBT)])

            @pl.when(s == SC_NS - 1)
            def _():
                pltpu.sync_copy(z_hbm.at[pl.ds(0, SPROWS - (SC_NS - 1) * WBT)],
                                seg.at[pl.ds((SC_NS - 1) * WBT,
                                             SPROWS - (SC_NS - 1) * WBT)])
            plsc.subcore_barrier()

            for b in (0, 1):
                pltpu.async_copy(
                    epk_hbm.at[pl.ds((s * nch + b) * 3 * ECHK, 3 * ECHK)],
                    epk_vm[b], seme[b])

            def pairc(j2, carry):
                for b in (0, 1):
                    j = j2 * 2 + b
                    base = j * ECHK
                    g = s * nch + j
                    pltpu.make_async_copy(
                        epk_hbm.at[pl.ds(g * 3 * ECHK, 3 * ECHK)],
                        epk_vm[b], seme[b]).wait()
                    for kk in range(ECHK // 16):
                        ksl = pl.ds(kk * 16, 16)
                        euv = epk_vm[b][pl.ds(kk * 16, 16)]
                        evv = epk_vm[b][pl.ds(ECHK + kk * 16, 16)]
                        etv = epk_vm[b][pl.ds(2 * ECHK + kk * 16, 16)]
                        du = euv * S + etv
                        dv = evv * S + etv
                        dst = du if uside else dv
                        src = dv if uside else du
                        m = (dst >= lo) & (dst < lo + segsz)
                        gstage[ksl] = src
                        sstage[ksl] = jnp.where(m, dst - lo, DUMMY)

                    @pl.when(j + 2 < nch)
                    def _():
                        pltpu.async_copy(
                            epk_hbm.at[pl.ds((g + 2) * 3 * ECHK, 3 * ECHK)],
                            epk_vm[b], seme[b])
                    gcp = pltpu.async_copy(table.at[gstage], rows_vm, sem)
                    pltpu.sync_copy(v16_hbm.at[pl.ds(ebase + base, ECHK)],
                                    v16_vm)
                    gcp.wait()

                    def scale(r4, carry2):
                        for dr in range(4):
                            r = r4 * 4 + dr
                            vv = v16_vm[r, pl.ds(0, 16)]
                            for kk in range(CP // 16):
                                ksl = pl.ds(kk * 16, 16)
                                rows_vm[r, ksl] = rows_vm[r, ksl] * vv
                        return carry2
                    lax.fori_loop(0, ECHK // 4, scale, 0)

                    pltpu.sync_copy(rows_vm, seg.at[sstage], add=True)
                return carry
            lax.fori_loop(0, nch // 2, pairc, 0)
            plsc.subcore_barrier()

            @pl.when(s < SC_NS - 1)
            def _():
                off = pl.multiple_of(lo + s * WBT, 8)
                pltpu.sync_copy(seg.at[pl.ds(s * WBT, WBT)],
                                zout.at[pl.ds(off, WBT)])

            rem = segsz - (SC_NS - 1) * WBT

            @pl.when(s == SC_NS - 1)
            def _():
                off = pl.multiple_of(lo + (SC_NS - 1) * WBT, 8)
                pltpu.sync_copy(seg.at[pl.ds((SC_NS - 1) * WBT, rem)],
                                zout.at[pl.ds(off, rem)])
            plsc.subcore_barrier()

    return ek(epk, val16, zrs, tmp_u, tmp_v)


GCHK = 128
GBPW = P_PAD // (SC_NC * SC_NS)


def _pair_gather(a_cat, emb_v, ui_pad, ii_pad):
    nch = GBPW // GCHK

    @functools.partial(
        pl.kernel,
        mesh=plsc.VectorSubcoreMesh(core_axis_name="c", subcore_axis_name="s"),
        out_type=[
            jax.ShapeDtypeStruct((P_PAD, 2 * H1P), f32),
            jax.ShapeDtypeStruct((P_PAD, H1P), f32),
        ],
        scratch_types=[
            pltpu.VMEM((GBPW,), jnp.int32),
            pltpu.VMEM((GBPW,), jnp.int32),
            pltpu.VMEM((GCHK, 2 * H1P), f32),
            pltpu.VMEM((GCHK, 2 * H1P), f32),
            pltpu.VMEM((GCHK, H1P), f32),
            pltpu.VMEM((GCHK, H1P), f32),
            pltpu.SemaphoreType.DMA,
            pltpu.SemaphoreType.DMA,
            pltpu.SemaphoreType.DMA,
            pltpu.SemaphoreType.DMA,
            pltpu.SemaphoreType.DMA,
            pltpu.SemaphoreType.DMA,
            pltpu.SemaphoreType.DMA,
            pltpu.SemaphoreType.DMA,
        ],
    )
    def gk(acat_hbm, embv_hbm, ui_hbm, ii_hbm, uh_hbm, vb_hbm,
           idxu, idxv, ru0, ru1, rv0, rv1,
           sgu0, sgu1, sgv0, sgv1, swu0, swu1, swv0, swv1):
        wid = lax.axis_index("s") * SC_NC + lax.axis_index("c")
        base = wid * GBPW
        ru = (ru0, ru1)
        rv = (rv0, rv1)
        sgu = (sgu0, sgu1)
        sgv = (sgv0, sgv1)
        swu = (swu0, swu1)
        swv = (swv0, swv1)
        pltpu.sync_copy(ui_hbm.at[pl.ds(base, GBPW)], idxu)
        pltpu.sync_copy(ii_hbm.at[pl.ds(base, GBPW)], idxv)

        def do_chunk(jj, b, gcur, wl):
            gu = pltpu.async_copy(
                acat_hbm.at[idxu.at[pl.ds(jj * GCHK, GCHK)]], ru[b], sgu[b])
            gv = pltpu.async_copy(
                embv_hbm.at[idxv.at[pl.ds(jj * GCHK, GCHK)]], rv[b], sgv[b])
            return (gu, gv)

        def pair(j2, carry):
            g = [None, None]
            for b in (0, 1):
                g[b] = do_chunk(j2 * 2 + b, b, None, None)
            wl = []
            for b in (0, 1):
                off = base + (j2 * 2 + b) * GCHK
                g[b][0].wait()
                wl.append(pltpu.async_copy(ru[b],
                                           uh_hbm.at[pl.ds(off, GCHK)],
                                           swu[b]))
                g[b][1].wait()
                wl.append(pltpu.async_copy(rv[b],
                                           vb_hbm.at[pl.ds(off, GCHK)],
                                           swv[b]))
            for wcp in wl:
                wcp.wait()
            return carry

        lax.fori_loop(0, nch // 2, pair, 0)
        g = do_chunk(nch - 1, 0, None, None)
        off = base + (nch - 1) * GCHK
        g[0].wait()
        pltpu.sync_copy(ru[0], uh_hbm.at[pl.ds(off, GCHK)])
        g[1].wait()
        pltpu.sync_copy(rv[0], vb_hbm.at[pl.ds(off, GCHK)])

    return gk(a_cat, emb_v, ui_pad, ii_pad)


def kernel(u_features, v_features, edge_index, edge_type, edge_val, labels,
           user_indices, item_indices, u_features_side, v_features_side,
           W, Wf_u, bf_u, Wf_v, bf_v, Wd_u, Wd_v, P_basis, a_coef):
    wpad = jnp.pad(W.reshape(IN, S, C), ((0, 0), (0, 0), (0, CP - C)))
    wpad = wpad.reshape(IN, H0P)
    def pad_wd(Wd):
        wg = jnp.pad(Wd[:H0].reshape(S, C, H1), ((0, 0), (0, CP - C),
                                                 (0, H1P - H1)))
        wf = jnp.pad(Wd[H0:], ((0, 0), (0, H1P - H1)))
        return wg.reshape(H0P, H1P), wf
    wd_u, wdf_u = pad_wd(Wd_u)
    wd_v, wdf_v = pad_wd(Wd_v)
    p0 = jnp.pad(P_basis[0], ((0, H1P - H1), (0, H1P - H1)))
    p1 = jnp.pad(P_basis[1], ((0, H1P - H1), (0, H1P - H1)))
    ac_pad = jnp.zeros((8, H1P), f32).at[:NB, :NC].set(a_coef)
    bf_u2 = bf_u.reshape(1, FH)
    bf_v2 = bf_v.reshape(1, FH)

    eu = jnp.pad(edge_index[0].astype(jnp.int32), (0, EPAD - E))
    ev = jnp.pad(edge_index[1].astype(jnp.int32), (0, EPAD - E))
    et = jnp.pad(edge_type.astype(jnp.int32), (0, EPAD - E))
    ew = jnp.pad(edge_val, (0, EPAD - E))
    ui = jnp.pad(user_indices.astype(jnp.int32), (0, P_PAD - P))
    ii = jnp.pad(item_indices.astype(jnp.int32), (0, P_PAD - P))
    lab2d = jnp.pad(labels.astype(jnp.int32), (0, P_PAD - P)).reshape(P_PAD, 1)

    tmp_u, feat_u = _pre(u_features, u_features_side, wpad, Wf_u, bf_u2)
    tmp_v, feat_v = _pre(v_features, v_features_side, wpad, Wf_v, bf_v2)

    val16 = jnp.broadcast_to(ew[:, None], (EPAD, 16))
    epk = jnp.stack([eu.reshape(EPAD // ECHK, ECHK),
                     ev.reshape(EPAD // ECHK, ECHK),
                     et.reshape(EPAD // ECHK, ECHK)], axis=1).reshape(-1)
    zrs = jnp.zeros((WBT, CP), f32)
    z_u, z_v = _edge_agg(tmp_u.reshape(U * S, CP), tmp_v.reshape(V * S, CP),
                         epk, val16, zrs)

    a_cat = _post_u(z_u.reshape(U, H0P), feat_u, wd_u, wdf_u, p0, p1)
    emb_v = _post_v(z_v.reshape(V, H0P), feat_v, wd_v, wdf_v)

    uh, vbm = _pair_gather(a_cat, emb_v, ui, ii)

    out8, loss11 = _decode(uh, vbm, lab2d, ac_pad)
    return out8[:P, :NC], loss11.reshape(())


# speedup vs baseline: 1.0025x; 1.0025x over previous
import functools

import jax
import jax.numpy as jnp
from jax import lax
from jax.experimental import pallas as pl
from jax.experimental.pallas import tpu as pltpu
from jax.experimental.pallas import tpu_sc as plsc

U = 10000
V = 10000
IN = 256
S = 5
H0 = 500
H1 = 75
C = 100
SIDE = 128
FH = 64
NB = 2
NC = 5
E = 160000
P = 100000

CP = 128
H0P = S * CP
H1P = 128

BR = 1000
f32 = jnp.float32

SC_NC = 2
SC_NS = 16


def _pre_body(x_ref, side_ref, wpad_ref, wf_ref, bf_ref, tmp_ref, feat_ref):
    tmp_ref[...] = jnp.dot(x_ref[...], wpad_ref[...],
                           preferred_element_type=f32)
    f = jnp.dot(side_ref[...], wf_ref[...], preferred_element_type=f32)
    feat_ref[...] = jnp.maximum(f + bf_ref[...], 0.0)


def _pre(x, side, wpad, wf, bf2d):
    n = x.shape[0]
    return pl.pallas_call(
        _pre_body,
        grid=(n // BR,),
        in_specs=[
            pl.BlockSpec((BR, IN), lambda i: (i, 0)),
            pl.BlockSpec((BR, SIDE), lambda i: (i, 0)),
            pl.BlockSpec((IN, H0P), lambda i: (0, 0)),
            pl.BlockSpec((SIDE, FH), lambda i: (0, 0)),
            pl.BlockSpec((1, FH), lambda i: (0, 0)),
        ],
        out_specs=[
            pl.BlockSpec((BR, H0P), lambda i: (i, 0)),
            pl.BlockSpec((BR, FH), lambda i: (i, 0)),
        ],
        out_shape=[
            jax.ShapeDtypeStruct((n, H0P), f32),
            jax.ShapeDtypeStruct((n, FH), f32),
        ],
    )(x, side, wpad, wf, bf2d)


def _post_u_body(z_ref, feat_ref, wd_ref, wdf_ref, p0_ref, p1_ref, out_ref):
    g = jnp.maximum(z_ref[...], 0.0)
    emb = (jnp.dot(g, wd_ref[...], preferred_element_type=f32)
           + jnp.dot(feat_ref[...], wdf_ref[...], preferred_element_type=f32))
    out_ref[:, :H1P] = jnp.dot(emb, p0_ref[...], preferred_element_type=f32)
    out_ref[:, H1P:] = jnp.dot(emb, p1_ref[...], preferred_element_type=f32)


def _post_u(z, feat, wd, wdf, p0, p1):
    return pl.pallas_call(
        _post_u_body,
        grid=(U // BR,),
        in_specs=[
            pl.BlockSpec((BR, H0P), lambda i: (i, 0)),
            pl.BlockSpec((BR, FH), lambda i: (i, 0)),
            pl.BlockSpec((H0P, H1P), lambda i: (0, 0)),
            pl.BlockSpec((FH, H1P), lambda i: (0, 0)),
            pl.BlockSpec((H1P, H1P), lambda i: (0, 0)),
            pl.BlockSpec((H1P, H1P), lambda i: (0, 0)),
        ],
        out_specs=[pl.BlockSpec((BR, 2 * H1P), lambda i: (i, 0))],
        out_shape=[jax.ShapeDtypeStruct((U, 2 * H1P), f32)],
    )(z, feat, wd, wdf, p0, p1)[0]


def _post_v_body(z_ref, feat_ref, wd_ref, wdf_ref, out_ref):
    g = jnp.maximum(z_ref[...], 0.0)
    out_ref[...] = (jnp.dot(g, wd_ref[...], preferred_element_type=f32)
                    + jnp.dot(feat_ref[...], wdf_ref[...],
                              preferred_element_type=f32))


def _post_v(z, feat, wd, wdf):
    return pl.pallas_call(
        _post_v_body,
        grid=(V // BR,),
        in_specs=[
            pl.BlockSpec((BR, H0P), lambda i: (i, 0)),
            pl.BlockSpec((BR, FH), lambda i: (i, 0)),
            pl.BlockSpec((H0P, H1P), lambda i: (0, 0)),
            pl.BlockSpec((FH, H1P), lambda i: (0, 0)),
        ],
        out_specs=[pl.BlockSpec((BR, H1P), lambda i: (i, 0))],
        out_shape=[jax.ShapeDtypeStruct((V, H1P), f32)],
    )(z, feat, wd, wdf)[0]


P_PAD = 102400
DBR = 1024


def _dec_body(uh_ref, vb_ref, lab_ref, ac_ref, out_ref, loss_ref):
    i = pl.program_id(0)
    uh = uh_ref[...]
    vb = vb_ref[...]
    d0 = jnp.sum(uh[:, :H1P] * vb, axis=1, keepdims=True)
    d1 = jnp.sum(uh[:, H1P:] * vb, axis=1, keepdims=True)
    logits = d0 * ac_ref[0:1, :] + d1 * ac_ref[1:2, :]
    lane = lax.broadcasted_iota(jnp.int32, logits.shape, 1)
    valid = lane < NC
    masked = jnp.where(valid, logits, -1e30)
    m = jnp.max(masked, axis=1, keepdims=True)
    se = jnp.sum(jnp.where(valid, jnp.exp(masked - m), 0.0),
                 axis=1, keepdims=True)
    lse = m + jnp.log(se)
    picked = jnp.sum(jnp.where(lane == lab_ref[...], logits, 0.0),
                     axis=1, keepdims=True)
    out_ref[...] = logits[:, :8]

    @pl.when(i == 0)
    def _():
        loss_ref[...] = jnp.zeros_like(loss_ref)

    row = i * DBR + lax.broadcasted_iota(jnp.int32, (DBR, 1), 0)
    contrib = jnp.where(row < P, lse - picked, 0.0)
    loss_ref[...] += jnp.sum(contrib)[None, None] / P


def _decode(uh, vb, lab2d, ac_pad):
    return pl.pallas_call(
        _dec_body,
        grid=(P_PAD // DBR,),
        in_specs=[
            pl.BlockSpec((DBR, 2 * H1P), lambda i: (i, 0)),
            pl.BlockSpec((DBR, H1P), lambda i: (i, 0)),
            pl.BlockSpec((DBR, 1), lambda i: (i, 0)),
            pl.BlockSpec((8, H1P), lambda i: (0, 0)),
        ],
        out_specs=[
            pl.BlockSpec((DBR, 8), lambda i: (i, 0)),
            pl.BlockSpec((1, 1), lambda i: (0, 0)),
        ],
        out_shape=[
            jax.ShapeDtypeStruct((P_PAD, 8), f32),
            jax.ShapeDtypeStruct((1, 1), f32),
        ],
    )(uh, vb, lab2d, ac_pad)


EPAD = 163840
EPT = EPAD // SC_NS
SEGA = 12504
SEGB = 12496
SPROWS = 12512
DUMMY = 12504
CCAP = EPT + 128
ZROWS = 8
ECHK = 80
WBT = 784


def _edge_agg(tmp_u, tmp_v, epk, val16, zrs):
    @functools.partial(
        pl.kernel,
        mesh=plsc.VectorSubcoreMesh(core_axis_name="c", subcore_axis_name="s"),
        out_type=[
            jax.ShapeDtypeStruct((U * S, CP), f32),
            jax.ShapeDtypeStruct((V * S, CP), f32),
        ],
        scratch_types=[
            pltpu.VMEM((3 * ECHK,), jnp.int32),
            pltpu.VMEM((3 * ECHK,), jnp.int32),
            pltpu.VMEM((ECHK, CP), f32),
            pltpu.VMEM((ECHK, 16), f32),
            pltpu.VMEM((ECHK,), jnp.int32),
            pltpu.VMEM((ECHK,), jnp.int32),
            pltpu.VMEM((ECHK,), jnp.int32),
            pltpu.VMEM_SHARED((SPROWS, CP), f32),
            pltpu.SemaphoreType.DMA,
            pltpu.SemaphoreType.DMA,
            pltpu.SemaphoreType.DMA,
            pltpu.SemaphoreType.DMA,
        ],
    )
    def ek(epk_hbm, v16_hbm, z_hbm, tu_hbm, tv_hbm, zu_hbm, zv_hbm,
           epk0, epk1, rows_vm, v16_vm, gstage, sstage0, sstage1, seg, sem,
           seme0, seme1, sems):
        epk_vm = (epk0, epk1)
        seme = (seme0, seme1)
        sstages = (sstage0, sstage1)
        c = lax.axis_index("c")
        s = lax.axis_index("s")
        ebase = s * EPT
        nch = EPT // ECHK

        for uside, k in ((True, 0), (True, 1), (False, 0), (False, 1)):
            table = tv_hbm if uside else tu_hbm
            zout = zu_hbm if uside else zv_hbm
            segsz = SEGA if k == 0 else SEGB
            lo = pl.multiple_of(c * 25000 + k * SEGA, 8)

            @pl.when(s < SC_NS - 1)
            def _():
                pltpu.sync_copy(z_hbm, seg.at[pl.ds(s * WBT, WBT)])

            @pl.when(s == SC_NS - 1)
            def _():
                pltpu.sync_copy(z_hbm.at[pl.ds(0, SPROWS - (SC_NS - 1) * WBT)],
                                seg.at[pl.ds((SC_NS - 1) * WBT,
                                             SPROWS - (SC_NS - 1) * WBT)])
            plsc.subcore_barrier()

            for b in (0, 1):
                pltpu.async_copy(
                    epk_hbm.at[pl.ds((s * nch + b) * 3 * ECHK, 3 * ECHK)],
                    epk_vm[b], seme[b])

            def pairc(j2, carry):
                scp = [None]
                for b in (0, 1):
                    j = j2 * 2 + b
                    base = j * ECHK
                    g = s * nch + j
                    pltpu.make_async_copy(
                        epk_hbm.at[pl.ds(g * 3 * ECHK, 3 * ECHK)],
                        epk_vm[b], seme[b]).wait()
                    for kk in range(ECHK // 16):
                        ksl = pl.ds(kk * 16, 16)
                        euv = epk_vm[b][pl.ds(kk * 16, 16)]
                        evv = epk_vm[b][pl.ds(ECHK + kk * 16, 16)]
                        etv = epk_vm[b][pl.ds(2 * ECHK + kk * 16, 16)]
                        du = euv * S + etv
                        dv = evv * S + etv
                        dst = du if uside else dv
                        src = dv if uside else du
                        m = (dst >= lo) & (dst < lo + segsz)
                        gstage[ksl] = src
                        sstages[b][ksl] = jnp.where(m, dst - lo, DUMMY)

                    @pl.when(j + 2 < nch)
                    def _():
                        pltpu.async_copy(
                            epk_hbm.at[pl.ds((g + 2) * 3 * ECHK, 3 * ECHK)],
                            epk_vm[b], seme[b])
                    if scp[0] is not None:
                        scp[0].wait()
                    gcp = pltpu.async_copy(table.at[gstage], rows_vm, sem)
                    pltpu.sync_copy(v16_hbm.at[pl.ds(ebase + base, ECHK)],
                                    v16_vm)
                    gcp.wait()

                    def scale(r4, carry2):
                        for dr in range(4):
                            r = r4 * 4 + dr
                            vv = v16_vm[r, pl.ds(0, 16)]
                            for kk in range(CP // 16):
                                ksl = pl.ds(kk * 16, 16)
                                rows_vm[r, ksl] = rows_vm[r, ksl] * vv
                        return carry2
                    lax.fori_loop(0, ECHK // 4, scale, 0)

                    if b == 0:
                        scp[0] = pltpu.async_copy(rows_vm, seg.at[sstages[0]],
                                                  sems, add=True)
                    else:
                        pltpu.sync_copy(rows_vm, seg.at[sstages[1]],
                                        add=True)
                return carry
            lax.fori_loop(0, nch // 2, pairc, 0)
            plsc.subcore_barrier()

            @pl.when(s < SC_NS - 1)
            def _():
                off = pl.multiple_of(lo + s * WBT, 8)
                pltpu.sync_copy(seg.at[pl.ds(s * WBT, WBT)],
                                zout.at[pl.ds(off, WBT)])

            rem = segsz - (SC_NS - 1) * WBT

            @pl.when(s == SC_NS - 1)
            def _():
                off = pl.multiple_of(lo + (SC_NS - 1) * WBT, 8)
                pltpu.sync_copy(seg.at[pl.ds((SC_NS - 1) * WBT, rem)],
                                zout.at[pl.ds(off, rem)])
            plsc.subcore_barrier()

    return ek(epk, val16, zrs, tmp_u, tmp_v)


GCHK = 128
GBPW = P_PAD // (SC_NC * SC_NS)


def _pair_gather(a_cat, emb_v, ui_pad, ii_pad):
    nch = GBPW // GCHK

    @functools.partial(
        pl.kernel,
        mesh=plsc.VectorSubcoreMesh(core_axis_name="c", subcore_axis_name="s"),
        out_type=[
            jax.ShapeDtypeStruct((P_PAD, 2 * H1P), f32),
            jax.ShapeDtypeStruct((P_PAD, H1P), f32),
        ],
        scratch_types=[
            pltpu.VMEM((GBPW,), jnp.int32),
            pltpu.VMEM((GBPW,), jnp.int32),
            pltpu.VMEM((GCHK, 2 * H1P), f32),
            pltpu.VMEM((GCHK, 2 * H1P), f32),
            pltpu.VMEM((GCHK, H1P), f32),
            pltpu.VMEM((GCHK, H1P), f32),
            pltpu.SemaphoreType.DMA,
            pltpu.SemaphoreType.DMA,
            pltpu.SemaphoreType.DMA,
            pltpu.SemaphoreType.DMA,
            pltpu.SemaphoreType.DMA,
            pltpu.SemaphoreType.DMA,
            pltpu.SemaphoreType.DMA,
            pltpu.SemaphoreType.DMA,
        ],
    )
    def gk(acat_hbm, embv_hbm, ui_hbm, ii_hbm, uh_hbm, vb_hbm,
           idxu, idxv, ru0, ru1, rv0, rv1,
           sgu0, sgu1, sgv0, sgv1, swu0, swu1, swv0, swv1):
        wid = lax.axis_index("s") * SC_NC + lax.axis_index("c")
        base = wid * GBPW
        ru = (ru0, ru1)
        rv = (rv0, rv1)
        sgu = (sgu0, sgu1)
        sgv = (sgv0, sgv1)
        swu = (swu0, swu1)
        swv = (swv0, swv1)
        pltpu.sync_copy(ui_hbm.at[pl.ds(base, GBPW)], idxu)
        pltpu.sync_copy(ii_hbm.at[pl.ds(base, GBPW)], idxv)

        def do_chunk(jj, b, gcur, wl):
            gu = pltpu.async_copy(
                acat_hbm.at[idxu.at[pl.ds(jj * GCHK, GCHK)]], ru[b], sgu[b])
            gv = pltpu.async_copy(
                embv_hbm.at[idxv.at[pl.ds(jj * GCHK, GCHK)]], rv[b], sgv[b])
            return (gu, gv)

        def pair(j2, carry):
            g = [None, None]
            for b in (0, 1):
                g[b] = do_chunk(j2 * 2 + b, b, None, None)
            wl = []
            for b in (0, 1):
                off = base + (j2 * 2 + b) * GCHK
                g[b][0].wait()
                wl.append(pltpu.async_copy(ru[b],
                                           uh_hbm.at[pl.ds(off, GCHK)],
                                           swu[b]))
                g[b][1].wait()
                wl.append(pltpu.async_copy(rv[b],
                                           vb_hbm.at[pl.ds(off, GCHK)],
                                           swv[b]))
            for wcp in wl:
                wcp.wait()
            return carry

        lax.fori_loop(0, nch // 2, pair, 0)
        g = do_chunk(nch - 1, 0, None, None)
        off = base + (nch - 1) * GCHK
        g[0].wait()
        pltpu.sync_copy(ru[0], uh_hbm.at[pl.ds(off, GCHK)])
        g[1].wait()
        pltpu.sync_copy(rv[0], vb_hbm.at[pl.ds(off, GCHK)])

    return gk(a_cat, emb_v, ui_pad, ii_pad)


def kernel(u_features, v_features, edge_index, edge_type, edge_val, labels,
           user_indices, item_indices, u_features_side, v_features_side,
           W, Wf_u, bf_u, Wf_v, bf_v, Wd_u, Wd_v, P_basis, a_coef):
    wpad = jnp.pad(W.reshape(IN, S, C), ((0, 0), (0, 0), (0, CP - C)))
    wpad = wpad.reshape(IN, H0P)
    def pad_wd(Wd):
        wg = jnp.pad(Wd[:H0].reshape(S, C, H1), ((0, 0), (0, CP - C),
                                                 (0, H1P - H1)))
        wf = jnp.pad(Wd[H0:], ((0, 0), (0, H1P - H1)))
        return wg.reshape(H0P, H1P), wf
    wd_u, wdf_u = pad_wd(Wd_u)
    wd_v, wdf_v = pad_wd(Wd_v)
    p0 = jnp.pad(P_basis[0], ((0, H1P - H1), (0, H1P - H1)))
    p1 = jnp.pad(P_basis[1], ((0, H1P - H1), (0, H1P - H1)))
    ac_pad = jnp.zeros((8, H1P), f32).at[:NB, :NC].set(a_coef)
    bf_u2 = bf_u.reshape(1, FH)
    bf_v2 = bf_v.reshape(1, FH)

    eu = jnp.pad(edge_index[0].astype(jnp.int32), (0, EPAD - E))
    ev = jnp.pad(edge_index[1].astype(jnp.int32), (0, EPAD - E))
    et = jnp.pad(edge_type.astype(jnp.int32), (0, EPAD - E))
    ew = jnp.pad(edge_val, (0, EPAD - E))
    ui = jnp.pad(user_indices.astype(jnp.int32), (0, P_PAD - P))
    ii = jnp.pad(item_indices.astype(jnp.int32), (0, P_PAD - P))
    lab2d = jnp.pad(labels.astype(jnp.int32), (0, P_PAD - P)).reshape(P_PAD, 1)

    tmp_u, feat_u = _pre(u_features, u_features_side, wpad, Wf_u, bf_u2)
    tmp_v, feat_v = _pre(v_features, v_features_side, wpad, Wf_v, bf_v2)

    val16 = jnp.broadcast_to(ew[:, None], (EPAD, 16))
    epk = jnp.stack([eu.reshape(EPAD // ECHK, ECHK),
                     ev.reshape(EPAD // ECHK, ECHK),
                     et.reshape(EPAD // ECHK, ECHK)], axis=1).reshape(-1)
    zrs = jnp.zeros((WBT, CP), f32)
    z_u, z_v = _edge_agg(tmp_u.reshape(U * S, CP), tmp_v.reshape(V * S, CP),
                         epk, val16, zrs)

    a_cat = _post_u(z_u.reshape(U, H0P), feat_u, wd_u, wdf_u, p0, p1)
    emb_v = _post_v(z_v.reshape(V, H0P), feat_v, wd_v, wdf_v)

    uh, vbm = _pair_gather(a_cat, emb_v, ui, ii)

    out8, loss11 = _decode(uh, vbm, lab2d, ac_pad)
    return out8[:P, :NC], loss11.reshape(())
